# baseline (speedup 1.0000x reference)
import functools

import jax
import jax.numpy as jnp
from jax.experimental import pallas as pl
from jax.experimental.pallas import tpu as pltpu

B, S, HID, FFN, E, TOPK = 2, 2048, 1024, 4096, 8, 2
T = B * S

TOK_BLK = 512
FFN_BLK = 512


def _router_body(x_ref, gw_ref, logits_ref, combine_ref):
    xb = x_ref[...]
    logits = jnp.dot(xb, gw_ref[...], preferred_element_type=jnp.float32)
    logits_ref[...] = logits
    m = jnp.max(logits, axis=1, keepdims=True)
    p = jnp.exp(logits - m)
    p = p / jnp.sum(p, axis=1, keepdims=True)
    e_iota = jax.lax.broadcasted_iota(jnp.int32, (TOK_BLK, E), 1)
    a1 = jnp.argmax(p, axis=1)
    oh1 = (e_iota == a1[:, None])
    p1 = jnp.max(p, axis=1, keepdims=True)
    p_masked = jnp.where(oh1, -jnp.inf, p)
    a2 = jnp.argmax(p_masked, axis=1)
    oh2 = (e_iota == a2[:, None])
    p2 = jnp.max(p_masked, axis=1, keepdims=True)
    denom = p1 + p2
    combine_ref[...] = jnp.where(oh1, p1 / denom, 0.0) + jnp.where(oh2, p2 / denom, 0.0)


def _router(x, gate_w):
    grid = (T // TOK_BLK,)
    return pl.pallas_call(
        _router_body,
        grid=grid,
        in_specs=[
            pl.BlockSpec((TOK_BLK, HID), lambda b: (b, 0)),
            pl.BlockSpec((HID, E), lambda b: (0, 0)),
        ],
        out_specs=[
            pl.BlockSpec((TOK_BLK, E), lambda b: (b, 0)),
            pl.BlockSpec((TOK_BLK, E), lambda b: (b, 0)),
        ],
        out_shape=[
            jax.ShapeDtypeStruct((T, E), jnp.float32),
            jax.ShapeDtypeStruct((T, E), jnp.float32),
        ],
    )(x, gate_w)


def _ffn_body(x_ref, comb_ref, w1_ref, w3_ref, w2_ref, out_ref):
    e = pl.program_id(1)
    f = pl.program_id(2)
    xb = x_ref[...]
    a = jnp.dot(xb, w1_ref[0], preferred_element_type=jnp.float32)
    g = jnp.dot(xb, w3_ref[0], preferred_element_type=jnp.float32)
    h = a * jax.nn.sigmoid(a) * g
    part = jnp.dot(h, w2_ref[0], preferred_element_type=jnp.float32)
    e_iota = jax.lax.broadcasted_iota(jnp.int32, (TOK_BLK, E), 1)
    cw = jnp.sum(jnp.where(e_iota == e, comb_ref[...], 0.0), axis=1, keepdims=True)
    contrib = part * cw

    @pl.when((e == 0) & (f == 0))
    def _init():
        out_ref[...] = contrib

    @pl.when((e > 0) | (f > 0))
    def _acc():
        out_ref[...] += contrib


def _ffn_dense(x, combine, W1, W3, W2):
    grid = (T // TOK_BLK, E, FFN // FFN_BLK)
    return pl.pallas_call(
        _ffn_body,
        grid=grid,
        in_specs=[
            pl.BlockSpec((TOK_BLK, HID), lambda b, e, f: (b, 0)),
            pl.BlockSpec((TOK_BLK, E), lambda b, e, f: (b, 0)),
            pl.BlockSpec((1, HID, FFN_BLK), lambda b, e, f: (e, 0, f)),
            pl.BlockSpec((1, HID, FFN_BLK), lambda b, e, f: (e, 0, f)),
            pl.BlockSpec((1, FFN_BLK, HID), lambda b, e, f: (e, f, 0)),
        ],
        out_specs=pl.BlockSpec((TOK_BLK, HID), lambda b, e, f: (b, 0)),
        out_shape=jax.ShapeDtypeStruct((T, HID), jnp.float32),
        compiler_params=pltpu.CompilerParams(
            dimension_semantics=("arbitrary", "arbitrary", "arbitrary"),
        ),
    )(x, combine, W1, W3, W2)


@functools.partial(jax.jit, static_argnames=())
def kernel(hidden_states, gate_w, W1, W3, W2):
    b, s, hid = hidden_states.shape
    x = hidden_states.reshape(-1, hid)
    router_logits, combine = _router(x, gate_w)
    final = _ffn_dense(x, combine, W1, W3, W2)
    return final.reshape(b, s, hid), router_logits


# speedup vs baseline: 1.6811x; 1.6811x over previous
import functools

import jax
import jax.numpy as jnp
from jax.experimental import pallas as pl
from jax.experimental.pallas import tpu as pltpu

B, S, HID, FFN, E, TOPK = 2, 2048, 1024, 4096, 8, 2
T = B * S
A = T * TOPK

BM = 512
NBLK = A // BM + E
NROWS = NBLK * BM
FFN_BLK = 512
TOK_BLK = 512


def _router_body(x_ref, gw_ref, logits_ref, w_ref, ids_ref):
    xb = x_ref[...]
    logits = jnp.dot(xb, gw_ref[...], preferred_element_type=jnp.float32)
    logits_ref[...] = logits
    m = jnp.max(logits, axis=1, keepdims=True)
    p = jnp.exp(logits - m)
    p = p / jnp.sum(p, axis=1, keepdims=True)
    e_iota = jax.lax.broadcasted_iota(jnp.int32, (TOK_BLK, E), 1)
    a1 = jnp.argmax(p, axis=1)
    oh1 = (e_iota == a1[:, None])
    p1 = jnp.max(p, axis=1)
    p_masked = jnp.where(oh1, -jnp.inf, p)
    a2 = jnp.argmax(p_masked, axis=1)
    p2 = jnp.max(p_masked, axis=1)
    denom = p1 + p2
    w_ref[...] = jnp.stack([p1 / denom, p2 / denom], axis=1)
    ids_ref[...] = jnp.stack([a1, a2], axis=1).astype(jnp.int32)


def _router(x, gate_w):
    grid = (T // TOK_BLK,)
    return pl.pallas_call(
        _router_body,
        grid=grid,
        in_specs=[
            pl.BlockSpec((TOK_BLK, HID), lambda b: (b, 0)),
            pl.BlockSpec((HID, E), lambda b: (0, 0)),
        ],
        out_specs=[
            pl.BlockSpec((TOK_BLK, E), lambda b: (b, 0)),
            pl.BlockSpec((TOK_BLK, TOPK), lambda b: (b, 0)),
            pl.BlockSpec((TOK_BLK, TOPK), lambda b: (b, 0)),
        ],
        out_shape=[
            jax.ShapeDtypeStruct((T, E), jnp.float32),
            jax.ShapeDtypeStruct((T, TOPK), jnp.float32),
            jax.ShapeDtypeStruct((T, TOPK), jnp.int32),
        ],
    )(x, gate_w)


def _gffn_body(be_ref, xg_ref, w1_ref, w3_ref, w2_ref, out_ref):
    f = pl.program_id(1)
    xb = xg_ref[...]
    a = jnp.dot(xb, w1_ref[0], preferred_element_type=jnp.float32)
    g = jnp.dot(xb, w3_ref[0], preferred_element_type=jnp.float32)
    h = a * jax.nn.sigmoid(a) * g
    part = jnp.dot(h, w2_ref[0], preferred_element_type=jnp.float32)

    @pl.when(f == 0)
    def _init():
        out_ref[...] = part

    @pl.when(f > 0)
    def _acc():
        out_ref[...] += part


def _gffn(xg, block_expert, W1, W3, W2):
    grid_spec = pltpu.PrefetchScalarGridSpec(
        num_scalar_prefetch=1,
        grid=(NBLK, FFN // FFN_BLK),
        in_specs=[
            pl.BlockSpec((BM, HID), lambda b, f, be: (b, 0)),
            pl.BlockSpec((1, HID, FFN_BLK), lambda b, f, be: (be[b], 0, f)),
            pl.BlockSpec((1, HID, FFN_BLK), lambda b, f, be: (be[b], 0, f)),
            pl.BlockSpec((1, FFN_BLK, HID), lambda b, f, be: (be[b], f, 0)),
        ],
        out_specs=pl.BlockSpec((BM, HID), lambda b, f, be: (b, 0)),
    )
    return pl.pallas_call(
        _gffn_body,
        grid_spec=grid_spec,
        out_shape=jax.ShapeDtypeStruct((NROWS, HID), jnp.float32),
        compiler_params=pltpu.CompilerParams(
            dimension_semantics=("arbitrary", "arbitrary"),
        ),
    )(block_expert, xg, W1, W3, W2)


@functools.partial(jax.jit, static_argnames=())
def kernel(hidden_states, gate_w, W1, W3, W2):
    b, s, hid = hidden_states.shape
    x = hidden_states.reshape(-1, hid)
    router_logits, w, ids = _router(x, gate_w)

    flat_ids = ids.reshape(-1)
    oh = (flat_ids[:, None] == jnp.arange(E, dtype=jnp.int32)[None, :])
    counts = jnp.sum(oh.astype(jnp.int32), axis=0)
    nblk_e = (counts + BM - 1) // BM
    blk_end = jnp.cumsum(nblk_e)
    pstart = (blk_end - nblk_e) * BM
    csum = jnp.cumsum(oh.astype(jnp.int32), axis=0)
    rank = jnp.take_along_axis(csum, flat_ids[:, None], axis=1)[:, 0] - 1
    pos = pstart[flat_ids] + rank
    blk_iota = jnp.arange(NBLK, dtype=jnp.int32)
    block_expert = jnp.sum(
        (blk_iota[:, None] >= blk_end[None, :]).astype(jnp.int32), axis=1)
    block_expert = jnp.minimum(block_expert, E - 1)
    src = jnp.zeros((NROWS,), jnp.int32).at[pos].set(
        jnp.arange(A, dtype=jnp.int32) // TOPK)

    xg = x[src]
    og = _gffn(xg, block_expert, W1, W3, W2)
    out_rows = og[pos]
    final = jnp.sum(out_rows.reshape(T, TOPK, hid) * w[..., None], axis=1)
    return final.reshape(b, s, hid), router_logits
